# baseline (speedup 1.0000x reference)
import jax
from jax.experimental import pallas as pl
from jax.experimental.pallas import tpu as pltpu

_BLOCK = 5000


def _copy_block(src_ref, out_ref):
    out_ref[...] = src_ref[...]


def kernel(mo_features, coupling_strengths, mo_neighbours_i, mo_neighbours_j,
           W_as1, b_as1, W_as2, b_as2, W_mx1, b_mx1, W_mx2, b_mx2):
    n, f = mo_features.shape
    return pl.pallas_call(
        _copy_block,
        grid=(n // _BLOCK,),
        in_specs=[pl.BlockSpec((_BLOCK, f), lambda i: (i, 0))],
        out_specs=pl.BlockSpec((_BLOCK, f), lambda i: (i, 0)),
        out_shape=jax.ShapeDtypeStruct((n, f), mo_features.dtype),
        compiler_params=pltpu.CompilerParams(dimension_semantics=("parallel",)),
    )(mo_features)
